# Initial kernel scaffold; baseline (speedup 1.0000x reference)
import jax
import jax.numpy as jnp
from jax.experimental import pallas as pl


def kernel(X_real, X_imag, L_norm_real, L_norm_imag, weight, bias):
    raise NotImplementedError("write your pallas kernel here")


# speedup vs baseline: 1.4239x; 1.4239x over previous
import jax
import jax.numpy as jnp
from jax.experimental import pallas as pl


def _sdconv_block(lr_ref, li_ref, xc_ref, wr_ref, wi_ref, b_ref, out_ref):
    xc = xc_ref[...]
    acc = jnp.broadcast_to(b_ref[...], out_ref.shape)
    for i in range(lr_ref.shape[0]):
        yr = jnp.dot(lr_ref[i], xc, preferred_element_type=jnp.float32)
        yi = jnp.dot(li_ref[i], xc, preferred_element_type=jnp.float32)
        acc = acc + jnp.dot(yr, wr_ref[i], preferred_element_type=jnp.float32)
        acc = acc + jnp.dot(yi, wi_ref[i], preferred_element_type=jnp.float32)
    out_ref[...] = acc


def kernel(X_real, X_imag, L_norm_real, L_norm_imag, weight, bias):
    N, D = X_real.shape
    Kp1, _, D_out = weight.shape

    xc = jnp.concatenate([X_real, X_imag], axis=1)
    z = jnp.zeros_like(weight)
    wr = jnp.concatenate(
        [jnp.concatenate([weight, z], axis=2),
         jnp.concatenate([z, weight], axis=2)], axis=1)
    wi = jnp.concatenate(
        [jnp.concatenate([z, weight], axis=2),
         jnp.concatenate([-weight, z], axis=2)], axis=1)
    b2 = jnp.concatenate([bias, bias], axis=1)

    BM = 256
    grid = (N // BM,)
    out = pl.pallas_call(
        _sdconv_block,
        grid=grid,
        in_specs=[
            pl.BlockSpec((Kp1, BM, N), lambda r: (0, r, 0)),
            pl.BlockSpec((Kp1, BM, N), lambda r: (0, r, 0)),
            pl.BlockSpec((N, 2 * D), lambda r: (0, 0)),
            pl.BlockSpec((Kp1, 2 * D, 2 * D_out), lambda r: (0, 0, 0)),
            pl.BlockSpec((Kp1, 2 * D, 2 * D_out), lambda r: (0, 0, 0)),
            pl.BlockSpec((1, 2 * D_out), lambda r: (0, 0)),
        ],
        out_specs=pl.BlockSpec((BM, 2 * D_out), lambda r: (r, 0)),
        out_shape=jax.ShapeDtypeStruct((N, 2 * D_out), jnp.float32),
    )(L_norm_real, L_norm_imag, xc, wr, wi, b2)

    real = out[:, :D_out]
    imag = out[:, D_out:]
    return (real, imag, L_norm_real, L_norm_imag)


# speedup vs baseline: 1.4242x; 1.0002x over previous
import jax
import jax.numpy as jnp
from jax.experimental import pallas as pl


def _sdconv_block(lr_ref, li_ref, xc_ref, wr_ref, wi_ref, b_ref, out_ref):
    xc = xc_ref[...].astype(jnp.bfloat16)
    acc = jnp.broadcast_to(b_ref[...], out_ref.shape)
    for i in range(lr_ref.shape[0]):
        yr = jnp.dot(lr_ref[i].astype(jnp.bfloat16), xc,
                     preferred_element_type=jnp.float32)
        yi = jnp.dot(li_ref[i].astype(jnp.bfloat16), xc,
                     preferred_element_type=jnp.float32)
        acc = acc + jnp.dot(yr.astype(jnp.bfloat16),
                            wr_ref[i].astype(jnp.bfloat16),
                            preferred_element_type=jnp.float32)
        acc = acc + jnp.dot(yi.astype(jnp.bfloat16),
                            wi_ref[i].astype(jnp.bfloat16),
                            preferred_element_type=jnp.float32)
    out_ref[...] = acc


def kernel(X_real, X_imag, L_norm_real, L_norm_imag, weight, bias):
    N, D = X_real.shape
    Kp1, _, D_out = weight.shape

    xc = jnp.concatenate([X_real, X_imag], axis=1)
    z = jnp.zeros_like(weight)
    wr = jnp.concatenate(
        [jnp.concatenate([weight, z], axis=2),
         jnp.concatenate([z, weight], axis=2)], axis=1)
    wi = jnp.concatenate(
        [jnp.concatenate([z, weight], axis=2),
         jnp.concatenate([-weight, z], axis=2)], axis=1)
    b2 = jnp.concatenate([bias, bias], axis=1)

    BM = 256
    grid = (N // BM,)
    out = pl.pallas_call(
        _sdconv_block,
        grid=grid,
        in_specs=[
            pl.BlockSpec((Kp1, BM, N), lambda r: (0, r, 0)),
            pl.BlockSpec((Kp1, BM, N), lambda r: (0, r, 0)),
            pl.BlockSpec((N, 2 * D), lambda r: (0, 0)),
            pl.BlockSpec((Kp1, 2 * D, 2 * D_out), lambda r: (0, 0, 0)),
            pl.BlockSpec((Kp1, 2 * D, 2 * D_out), lambda r: (0, 0, 0)),
            pl.BlockSpec((1, 2 * D_out), lambda r: (0, 0)),
        ],
        out_specs=pl.BlockSpec((BM, 2 * D_out), lambda r: (r, 0)),
        out_shape=jax.ShapeDtypeStruct((N, 2 * D_out), jnp.float32),
    )(L_norm_real, L_norm_imag, xc, wr, wi, b2)

    real = out[:, :D_out]
    imag = out[:, D_out:]
    return (real, imag, L_norm_real, L_norm_imag)
